# Initial kernel scaffold; baseline (speedup 1.0000x reference)
import jax
import jax.numpy as jnp
from jax.experimental import pallas as pl


def kernel(x, tables):
    raise NotImplementedError("write your pallas kernel here")


# speedup vs baseline: 1.2014x; 1.2014x over previous
import functools

import jax
import jax.numpy as jnp
from jax import lax
from jax.experimental import pallas as pl
from jax.experimental.pallas import tpu as pltpu
from jax.experimental.pallas import tpu_sc as plsc

N_FIELDS = 26
VOCAB = 100000
EMB_DIM = 32
BATCH = 16384

ROWS = BATCH * N_FIELDS
NUM_CORES = 2
NUM_SUBCORES = 16
NW = NUM_CORES * NUM_SUBCORES
ROWS_PER_W = ROWS // NW
IDX_W = 128
G = 8
BLOCK = G * IDX_W
NBLK = ROWS_PER_W // BLOCK


def _make_gather():
    mesh = plsc.VectorSubcoreMesh(core_axis_name="c", subcore_axis_name="s")

    @functools.partial(
        pl.kernel,
        mesh=mesh,
        compiler_params=pltpu.CompilerParams(use_tc_tiling_on_sc=False),
        out_type=jax.ShapeDtypeStruct((ROWS, EMB_DIM), jnp.float32),
        scratch_types=[
            pltpu.VMEM((G, IDX_W), jnp.int32),
            pltpu.VMEM((BLOCK, EMB_DIM), jnp.float32),
            pltpu.SemaphoreType.DMA,
        ],
    )
    def gather_kernel(tab_hbm, idx_hbm, out_hbm, idx_v, rows_v, sem):
        wid = lax.axis_index("s") * NUM_CORES + lax.axis_index("c")
        row0 = wid * ROWS_PER_W
        iblk0 = row0 // IDX_W

        def body(blk, carry):
            base = row0 + blk * BLOCK
            irow = pl.multiple_of(iblk0 + blk * G, 8)
            pltpu.sync_copy(idx_hbm.at[pl.ds(irow, G), :], idx_v)
            handles = [
                pltpu.async_copy(
                    tab_hbm.at[idx_v.at[g]],
                    rows_v.at[pl.ds(g * IDX_W, IDX_W)],
                    sem,
                )
                for g in range(G)
            ]
            for h in handles:
                h.wait()
            pltpu.sync_copy(rows_v, out_hbm.at[pl.ds(base, BLOCK)])
            return carry

        lax.fori_loop(0, NBLK, body, 0)

    return gather_kernel


_gather = _make_gather()


@jax.jit
def kernel(x, tables):
    offs = jnp.arange(N_FIELDS, dtype=jnp.int32) * VOCAB
    flat_idx = (x.astype(jnp.int32) + offs[None, :]).reshape(ROWS // IDX_W, IDX_W)
    flat_tab = tables.reshape(N_FIELDS * VOCAB, EMB_DIM)
    out = _gather(flat_tab, flat_idx)
    return out.reshape(BATCH, N_FIELDS * EMB_DIM)
